# baseline (speedup 1.0000x reference)
import functools

import jax
import jax.numpy as jnp
from jax import lax
from jax.experimental import pallas as pl
from jax.experimental.pallas import tpu as pltpu
from jax.experimental.pallas import tpu_sc as plsc

N = 10000
E = 320000
D = 128
H = 64
NC = 2
NS = 16
NP = 10240
BT = 1024
G = NP // BT
EP = E // NS
C = 80
NCHUNK = EP // C
RP = NP // NS


K = 5
KC = K * C
NG = NCHUNK // K
RC = RP // C


def _make_sc_body(first):
  def body(xcat, src2d, dst2d, *rest):
    if first:
      mean_out, inv_out, srcb, dstb, rows_v, ones_v, cntb, ivb, drainb, \
          agg_sh, cnt_sh, gsem, ssem, osem, isem = rest
    else:
      inv_in, mean_out, srcb, dstb, rows_v, ones_v, cntb, ivb, drainb, \
          agg_sh, cnt_sh, gsem, ssem, osem, isem = rest
    c = lax.axis_index("c")
    s = lax.axis_index("s")
    row0 = s * RP

    z16 = jnp.zeros((16,), jnp.float32)
    o16 = jnp.ones((16,), jnp.float32)

    def zrow(i, carry):
      for k in range(H // 16):
        rows_v[0, i, pl.ds(k * 16, 16)] = z16
      return carry
    lax.fori_loop(0, C, zrow, 0)
    for k in range(C // 16):
      cntb[pl.ds(k * 16, 16)] = z16
      ones_v[pl.ds(k * 16, 16)] = o16
    for r in range(RC):
      pltpu.sync_copy(rows_v.at[0, pl.ds(0, C)],
                      agg_sh.at[pl.ds(row0 + r * C, C)])
      if first:
        pltpu.sync_copy(cntb, cnt_sh.at[pl.ds(row0 + r * C, C)])

    plsc.subcore_barrier()

    def load_idx(g, p):
      d1 = pltpu.async_copy(src2d.at[c, s, pl.ds(g * K, K)], srcb.at[p],
                            isem)
      d2 = pltpu.async_copy(dst2d.at[s, pl.ds(g * K, K)], dstb.at[p], isem)
      return d1, d2

    def fire(p):
      for k in range(K):
        pltpu.async_copy(xcat.at[srcb.at[p, k]],
                         rows_v.at[p, pl.ds(k * C, C)], gsem)

    d1, d2 = load_idx(0, 0)
    d1.wait()
    d2.wait()
    fire(0)

    def gbody(g, carry):
      p = lax.rem(g, 2)
      q = 1 - p

      pltpu.make_async_copy(xcat.at[pl.ds(0, KC)], rows_v.at[p],
                            gsem).wait()
      for k in range(K):
        pltpu.async_copy(rows_v.at[p, pl.ds(k * C, C)],
                         agg_sh.at[dstb.at[p, k]], ssem, add=True)
        if first:
          pltpu.async_copy(ones_v, cnt_sh.at[dstb.at[p, k]], osem,
                           add=True)

      @pl.when(g > 0)
      def _():
        pltpu.make_async_copy(xcat.at[pl.ds(0, KC)], rows_v.at[q],
                              ssem).wait()
        if first:
          pltpu.make_async_copy(inv_out.at[pl.ds(0, KC)], drainb,
                                osem).wait()

      @pl.when(g + 1 < NG)
      def _():
        load_idx(g + 1, q)

      @pl.when(g + 1 < NG)
      def _():
        pltpu.make_async_copy(src2d.at[c, s, pl.ds(0, K)], srcb.at[q],
                              isem).wait()
        pltpu.make_async_copy(dst2d.at[s, pl.ds(0, K)], dstb.at[q],
                              isem).wait()
        fire(q)
      return carry
    lax.fori_loop(0, NG, gbody, 0)

    pfin = lax.rem(NG - 1, 2)
    pltpu.make_async_copy(xcat.at[pl.ds(0, KC)], rows_v.at[pfin],
                          ssem).wait()
    if first:
      pltpu.make_async_copy(inv_out.at[pl.ds(0, KC)], drainb,
                            osem).wait()
    plsc.subcore_barrier()

    def sbody(r, carry):
      base = row0 + r * C
      pltpu.sync_copy(agg_sh.at[pl.ds(base, C)], rows_v.at[0, pl.ds(0, C)])
      if first:
        pltpu.sync_copy(cnt_sh.at[pl.ds(base, C)], cntb)
        for k in range(C // 16):
          ivb[pl.ds(k * 16, 16)] = 1.0 / jnp.maximum(
              cntb[pl.ds(k * 16, 16)], 1.0)

        @pl.when(c == 0)
        def _():
          pltpu.sync_copy(ivb.at[pl.ds(0, C)], inv_out.at[pl.ds(base, C)])
      else:
        pltpu.sync_copy(inv_in.at[pl.ds(base, C)], ivb.at[pl.ds(0, C)])

      def srow(i, carry2):
        iv = ivb[pl.ds(i, 16)][0]
        for k in range(H // 16):
          rows_v[0, i, pl.ds(k * 16, 16)] = rows_v[0, i, pl.ds(k * 16, 16)] * iv
        return carry2
      lax.fori_loop(0, C, srow, 0)
      pltpu.sync_copy(rows_v.at[0, pl.ds(0, C)],
                      mean_out.at[c, pl.ds(base, C)])
      return carry
    lax.fori_loop(0, RC, sbody, 0)

  return body


def _make_sc_kernel(first):
  if first:
    out_type = [jax.ShapeDtypeStruct((NC, NP, H), jnp.float32),
                jax.ShapeDtypeStruct((NP,), jnp.float32)]
  else:
    out_type = jax.ShapeDtypeStruct((NC, NP, H), jnp.float32)
  return pl.kernel(
      _make_sc_body(first),
      out_type=out_type,
      mesh=plsc.VectorSubcoreMesh(
          core_axis_name="c", subcore_axis_name="s",
          num_cores=NC, num_subcores=NS),
      scratch_types=[
          pltpu.VMEM((2, K, C), jnp.int32),
          pltpu.VMEM((2, K, C), jnp.int32),
          pltpu.VMEM((2, KC, H), jnp.float32),
          pltpu.VMEM((C,), jnp.float32),
          pltpu.VMEM((C,), jnp.float32),
          pltpu.VMEM((C + 16,), jnp.float32),
          pltpu.VMEM((KC,), jnp.float32),
          pltpu.VMEM_SHARED((NP, H), jnp.float32),
          pltpu.VMEM_SHARED((NP,), jnp.float32),
          pltpu.SemaphoreType.DMA,
          pltpu.SemaphoreType.DMA,
          pltpu.SemaphoreType.DMA,
          pltpu.SemaphoreType.DMA,
      ],
      compiler_params=pltpu.CompilerParams(use_tc_tiling_on_sc=False),
  )


_sc_agg_first = _make_sc_kernel(True)
_sc_agg_next = _make_sc_kernel(False)


def _layer_norm_silu(h, g, be):
  mu = jnp.mean(h, axis=-1, keepdims=True)
  d = h - mu
  var = jnp.mean(d * d, axis=-1, keepdims=True)
  hn = d * lax.rsqrt(var + 1e-5) * g + be
  return hn * jax.nn.sigmoid(hn)


def _stage1_body(meanp_ref, x_ref, Wl_ref, Wr_ref, b_ref, g_ref, be_ref,
                 out_ref):
  mean = jnp.concatenate([meanp_ref[0], meanp_ref[1]], axis=-1)
  h = (jnp.dot(mean, Wl_ref[...], preferred_element_type=jnp.float32)
       + jnp.dot(x_ref[...], Wr_ref[...], preferred_element_type=jnp.float32)
       + b_ref[...])
  sil = _layer_norm_silu(h, g_ref[...], be_ref[...])
  out_ref[0] = sil[:, :H]
  out_ref[1] = sil[:, H:]


def _stage2_body(meanp_ref, hp_ref, Wl_ref, Wr_ref, b_ref, g_ref, be_ref,
                 Wm1_ref, bm1_ref, Wm2_ref, bm2_ref, out_ref):
  mean = jnp.concatenate([meanp_ref[0], meanp_ref[1]], axis=-1)
  hprev = jnp.concatenate([hp_ref[0], hp_ref[1]], axis=-1)
  h = (jnp.dot(mean, Wl_ref[...], preferred_element_type=jnp.float32)
       + jnp.dot(hprev, Wr_ref[...], preferred_element_type=jnp.float32)
       + b_ref[...])
  sil = _layer_norm_silu(h, g_ref[...], be_ref[...])
  m = jnp.maximum(
      jnp.dot(sil, Wm1_ref[...], preferred_element_type=jnp.float32)
      + bm1_ref[...], 0.0)
  out_ref[...] = (jnp.dot(m, Wm2_ref[...], preferred_element_type=jnp.float32)
                  + bm2_ref[...])


_half_spec = pl.BlockSpec((NC, BT, H), lambda i: (0, i, 0))
_full_spec = pl.BlockSpec((BT, D), lambda i: (i, 0))
_w_spec = pl.BlockSpec((D, D), lambda i: (0, 0))
_v_spec = pl.BlockSpec((1, D), lambda i: (0, 0))

_stage1 = pl.pallas_call(
    _stage1_body,
    grid=(G,),
    in_specs=[_half_spec, _full_spec, _w_spec, _w_spec,
              _v_spec, _v_spec, _v_spec],
    out_specs=_half_spec,
    out_shape=jax.ShapeDtypeStruct((NC, NP, H), jnp.float32),
)

_stage2 = pl.pallas_call(
    _stage2_body,
    grid=(G,),
    in_specs=[_half_spec, _half_spec, _w_spec, _w_spec,
              _v_spec, _v_spec, _v_spec,
              _w_spec, _v_spec, _w_spec, _v_spec],
    out_specs=pl.BlockSpec((BT, D), lambda i: (i, 0)),
    out_shape=jax.ShapeDtypeStruct((NP, D), jnp.float32),
)


@jax.jit
def kernel(x, edge_index, Wl0, Wr0, b0, g0, be0, Wl1, Wr1, b1, g1, be1,
           Wm1, bm1, Wm2, bm2):
  ei = edge_index.astype(jnp.int32)
  src = ei[0]
  dst = ei[1]
  xp = jnp.pad(x, ((0, NP - N), (0, 0)))
  xcat = jnp.concatenate([xp[:, :H], xp[:, H:]], axis=0)

  src2d = jnp.stack([src, src + NP]).reshape(NC, NS, NCHUNK, C)
  dst2d = dst.reshape(NS, NCHUNK, C)
  meanp0, inv = _sc_agg_first(xcat, src2d, dst2d)
  hp = _stage1(meanp0, xp, Wl0, Wr0,
               b0.reshape(1, D), g0.reshape(1, D), be0.reshape(1, D))
  hcat = hp.reshape(NC * NP, H)
  meanp1 = _sc_agg_next(hcat, src2d, dst2d, inv)
  Wm2p = jnp.pad(Wm2, ((0, 0), (0, D - 1)))
  bm2p = jnp.pad(bm2, (0, D - 1)).reshape(1, D)
  outp = _stage2(meanp1, hp, Wl1, Wr1,
                 b1.reshape(1, D), g1.reshape(1, D), be1.reshape(1, D),
                 Wm1, bm1.reshape(1, D), Wm2p, bm2p)
  return outp[:N, :1]


# speedup vs baseline: 1.0456x; 1.0456x over previous
import functools

import jax
import jax.numpy as jnp
from jax import lax
from jax.experimental import pallas as pl
from jax.experimental.pallas import tpu as pltpu
from jax.experimental.pallas import tpu_sc as plsc

N = 10000
E = 320000
D = 128
H = 64
NC = 2
NS = 16
NP = 10240
BT = 1000
G = N // BT
EP = E // NS
C = 80
NCHUNK = EP // C
RP = NP // NS


K = 5
KC = K * C
NG = NCHUNK // K
RC = RP // C


def _make_sc_body(first):
  def body(xlo, xhi, src2d, dst2d, *rest):
    if first:
      mean_out, inv_out, srcb, dstb, rows_v, ones_v, cntb, ivb, drainb, \
          agg_sh, cnt_sh, gsem, ssem, osem, isem = rest
    else:
      inv_in, mean_out, srcb, dstb, rows_v, ones_v, cntb, ivb, drainb, \
          agg_sh, cnt_sh, gsem, ssem, osem, isem = rest
    c = lax.axis_index("c")
    s = lax.axis_index("s")
    row0 = s * RP

    z16 = jnp.zeros((16,), jnp.float32)
    o16 = jnp.ones((16,), jnp.float32)

    def zrow(i, carry):
      for k in range(H // 16):
        rows_v[0, i, pl.ds(k * 16, 16)] = z16
      return carry
    lax.fori_loop(0, C, zrow, 0)
    for k in range(C // 16):
      cntb[pl.ds(k * 16, 16)] = z16
      ones_v[pl.ds(k * 16, 16)] = o16
    for r in range(RC):
      pltpu.sync_copy(rows_v.at[0, pl.ds(0, C)],
                      agg_sh.at[pl.ds(row0 + r * C, C)])
      if first:
        pltpu.sync_copy(cntb, cnt_sh.at[pl.ds(row0 + r * C, C)])

    plsc.subcore_barrier()

    def load_idx(g, p):
      d1 = pltpu.async_copy(src2d.at[s, pl.ds(g * K, K)], srcb.at[p], isem)
      d2 = pltpu.async_copy(dst2d.at[s, pl.ds(g * K, K)], dstb.at[p], isem)
      return d1, d2

    def fire(p):
      @pl.when(c == 0)
      def _():
        for k in range(K):
          pltpu.async_copy(xlo.at[srcb.at[p, k]],
                           rows_v.at[p, pl.ds(k * C, C)], gsem)

      @pl.when(c == 1)
      def _():
        for k in range(K):
          pltpu.async_copy(xhi.at[srcb.at[p, k]],
                           rows_v.at[p, pl.ds(k * C, C)], gsem)

    d1, d2 = load_idx(0, 0)
    d1.wait()
    d2.wait()
    fire(0)

    def gbody(g, carry):
      p = lax.rem(g, 2)
      q = 1 - p

      pltpu.make_async_copy(xlo.at[pl.ds(0, KC)], rows_v.at[p],
                            gsem).wait()
      for k in range(K):
        pltpu.async_copy(rows_v.at[p, pl.ds(k * C, C)],
                         agg_sh.at[dstb.at[p, k]], ssem, add=True)
        if first:
          pltpu.async_copy(ones_v, cnt_sh.at[dstb.at[p, k]], osem,
                           add=True)

      @pl.when(g > 0)
      def _():
        pltpu.make_async_copy(xlo.at[pl.ds(0, KC)], rows_v.at[q],
                              ssem).wait()
        if first:
          pltpu.make_async_copy(inv_out.at[pl.ds(0, KC)], drainb,
                                osem).wait()

      @pl.when(g + 1 < NG)
      def _():
        load_idx(g + 1, q)

      @pl.when(g + 1 < NG)
      def _():
        pltpu.make_async_copy(src2d.at[s, pl.ds(0, K)], srcb.at[q],
                              isem).wait()
        pltpu.make_async_copy(dst2d.at[s, pl.ds(0, K)], dstb.at[q],
                              isem).wait()
        fire(q)
      return carry
    lax.fori_loop(0, NG, gbody, 0)

    pfin = lax.rem(NG - 1, 2)
    pltpu.make_async_copy(xlo.at[pl.ds(0, KC)], rows_v.at[pfin],
                          ssem).wait()
    if first:
      pltpu.make_async_copy(inv_out.at[pl.ds(0, KC)], drainb,
                            osem).wait()
    plsc.subcore_barrier()

    def sbody(r, carry):
      base = row0 + r * C
      pltpu.sync_copy(agg_sh.at[pl.ds(base, C)], rows_v.at[0, pl.ds(0, C)])
      if first:
        pltpu.sync_copy(cnt_sh.at[pl.ds(base, C)], cntb)
        for k in range(C // 16):
          ivb[pl.ds(k * 16, 16)] = 1.0 / jnp.maximum(
              cntb[pl.ds(k * 16, 16)], 1.0)

        @pl.when(c == 0)
        def _():
          pltpu.sync_copy(ivb.at[pl.ds(0, C)], inv_out.at[pl.ds(base, C)])
      else:
        pltpu.sync_copy(inv_in.at[pl.ds(base, C)], ivb.at[pl.ds(0, C)])

      def srow(i, carry2):
        iv = ivb[pl.ds(i, 16)][0]
        for k in range(H // 16):
          rows_v[0, i, pl.ds(k * 16, 16)] = rows_v[0, i, pl.ds(k * 16, 16)] * iv
        return carry2
      lax.fori_loop(0, C, srow, 0)
      pltpu.sync_copy(rows_v.at[0, pl.ds(0, C)],
                      mean_out.at[c, pl.ds(base, C)])
      return carry
    lax.fori_loop(0, RC, sbody, 0)

  return body


def _make_sc_kernel(first):
  if first:
    out_type = [jax.ShapeDtypeStruct((NC, NP, H), jnp.float32),
                jax.ShapeDtypeStruct((NP,), jnp.float32)]
  else:
    out_type = jax.ShapeDtypeStruct((NC, NP, H), jnp.float32)
  return pl.kernel(
      _make_sc_body(first),
      out_type=out_type,
      mesh=plsc.VectorSubcoreMesh(
          core_axis_name="c", subcore_axis_name="s",
          num_cores=NC, num_subcores=NS),
      scratch_types=[
          pltpu.VMEM((2, K, C), jnp.int32),
          pltpu.VMEM((2, K, C), jnp.int32),
          pltpu.VMEM((2, KC, H), jnp.float32),
          pltpu.VMEM((C,), jnp.float32),
          pltpu.VMEM((C,), jnp.float32),
          pltpu.VMEM((C + 16,), jnp.float32),
          pltpu.VMEM((KC,), jnp.float32),
          pltpu.VMEM_SHARED((NP, H), jnp.float32),
          pltpu.VMEM_SHARED((NP,), jnp.float32),
          pltpu.SemaphoreType.DMA,
          pltpu.SemaphoreType.DMA,
          pltpu.SemaphoreType.DMA,
          pltpu.SemaphoreType.DMA,
      ],
      compiler_params=pltpu.CompilerParams(use_tc_tiling_on_sc=False),
  )


_sc_agg_first = _make_sc_kernel(True)
_sc_agg_next = _make_sc_kernel(False)


def _layer_norm_silu(h, g, be):
  mu = jnp.mean(h, axis=-1, keepdims=True)
  d = h - mu
  var = jnp.mean(d * d, axis=-1, keepdims=True)
  hn = d * lax.rsqrt(var + 1e-5) * g + be
  return hn * jax.nn.sigmoid(hn)


def _stage1_body(meanp_ref, x_ref, Wl_ref, Wr_ref, b_ref, g_ref, be_ref,
                 out_ref):
  mean = jnp.concatenate([meanp_ref[0], meanp_ref[1]], axis=-1)
  h = (jnp.dot(mean, Wl_ref[...], preferred_element_type=jnp.float32)
       + jnp.dot(x_ref[...], Wr_ref[...], preferred_element_type=jnp.float32)
       + b_ref[...])
  sil = _layer_norm_silu(h, g_ref[...], be_ref[...])
  out_ref[0] = sil[:, :H]
  out_ref[1] = sil[:, H:]


def _stage2_body(meanp_ref, hp_ref, Wl_ref, Wr_ref, b_ref, g_ref, be_ref,
                 Wm1_ref, bm1_ref, Wm2_ref, bm2_ref, out_ref):
  mean = jnp.concatenate([meanp_ref[0], meanp_ref[1]], axis=-1)
  hprev = jnp.concatenate([hp_ref[0], hp_ref[1]], axis=-1)
  h = (jnp.dot(mean, Wl_ref[...], preferred_element_type=jnp.float32)
       + jnp.dot(hprev, Wr_ref[...], preferred_element_type=jnp.float32)
       + b_ref[...])
  sil = _layer_norm_silu(h, g_ref[...], be_ref[...])
  m = jnp.maximum(
      jnp.dot(sil, Wm1_ref[...], preferred_element_type=jnp.float32)
      + bm1_ref[...], 0.0)
  out_ref[...] = (jnp.dot(m, Wm2_ref[...], preferred_element_type=jnp.float32)
                  + bm2_ref[...])


_half_spec = pl.BlockSpec((NC, BT, H), lambda i: (0, i, 0))
_full_spec = pl.BlockSpec((BT, D), lambda i: (i, 0))
_w_spec = pl.BlockSpec((D, D), lambda i: (0, 0))
_v_spec = pl.BlockSpec((1, D), lambda i: (0, 0))
_w2_spec = pl.BlockSpec((D, 1), lambda i: (0, 0))
_v2_spec = pl.BlockSpec((1, 1), lambda i: (0, 0))

_stage1 = pl.pallas_call(
    _stage1_body,
    grid=(G,),
    in_specs=[_half_spec, _full_spec, _w_spec, _w_spec,
              _v_spec, _v_spec, _v_spec],
    out_specs=_half_spec,
    out_shape=jax.ShapeDtypeStruct((NC, N, H), jnp.float32),
)

_stage2 = pl.pallas_call(
    _stage2_body,
    grid=(G,),
    in_specs=[_half_spec, _half_spec, _w_spec, _w_spec,
              _v_spec, _v_spec, _v_spec,
              _w_spec, _v_spec, _w2_spec, _v2_spec],
    out_specs=pl.BlockSpec((BT, 1), lambda i: (i, 0)),
    out_shape=jax.ShapeDtypeStruct((N, 1), jnp.float32),
)


@jax.jit
def kernel(x, edge_index, Wl0, Wr0, b0, g0, be0, Wl1, Wr1, b1, g1, be1,
           Wm1, bm1, Wm2, bm2):
  ei = edge_index.astype(jnp.int32)
  src2d = ei[0].reshape(NS, NCHUNK, C)
  dst2d = ei[1].reshape(NS, NCHUNK, C)

  meanp0, inv = _sc_agg_first(x[:, :H], x[:, H:], src2d, dst2d)
  hp = _stage1(meanp0, x, Wl0, Wr0,
               b0.reshape(1, D), g0.reshape(1, D), be0.reshape(1, D))
  meanp1 = _sc_agg_next(hp[0], hp[1], src2d, dst2d, inv)
  return _stage2(meanp1, hp, Wl1, Wr1,
                 b1.reshape(1, D), g1.reshape(1, D), be1.reshape(1, D),
                 Wm1, bm1.reshape(1, D), Wm2, bm2.reshape(1, 1))


# speedup vs baseline: 1.0659x; 1.0195x over previous
import functools

import jax
import jax.numpy as jnp
from jax import lax
from jax.experimental import pallas as pl
from jax.experimental.pallas import tpu as pltpu
from jax.experimental.pallas import tpu_sc as plsc

N = 10000
E = 320000
D = 128
H = 64
NC = 2
NS = 16
NP = 10240
BT = 1000
G = N // BT
EP = E // NS
C = 80
NCHUNK = EP // C
RP = NP // NS


K = 5
KC = K * C
NG = NCHUNK // K
RC = RP // C


def _make_sc_body(first):
  def body(xtab, src2d, dst2d, *rest):
    if first:
      mean_out, inv_out, srcb, dstb, rows_v, ones_v, cntb, ivb, drainb, \
          agg_sh, cnt_sh, gsem, ssem, osem, isem = rest
    else:
      inv_in, mean_out, srcb, dstb, rows_v, ones_v, cntb, ivb, drainb, \
          agg_sh, cnt_sh, gsem, ssem, osem, isem = rest
    c = lax.axis_index("c")
    s = lax.axis_index("s")
    row0 = s * RP

    z16 = jnp.zeros((16,), jnp.float32)
    o16 = jnp.ones((16,), jnp.float32)

    def zrow(i, carry):
      for k in range(H // 16):
        rows_v[0, i, pl.ds(k * 16, 16)] = z16
      return carry
    lax.fori_loop(0, C, zrow, 0)
    for k in range(C // 16):
      cntb[pl.ds(k * 16, 16)] = z16
      ones_v[pl.ds(k * 16, 16)] = o16
    for r in range(RC):
      pltpu.sync_copy(rows_v.at[0, pl.ds(0, C)],
                      agg_sh.at[pl.ds(row0 + r * C, C)])
      if first:
        pltpu.sync_copy(cntb, cnt_sh.at[pl.ds(row0 + r * C, C)])

    plsc.subcore_barrier()

    def load_idx(g, p):
      d1 = pltpu.async_copy(src2d.at[s, pl.ds(g * K, K)], srcb.at[p], isem)
      d2 = pltpu.async_copy(dst2d.at[s, pl.ds(g * K, K)], dstb.at[p], isem)
      return d1, d2

    def fire(p):
      for k in range(K):
        pltpu.async_copy(xtab.at[srcb.at[p, k]],
                         rows_v.at[p, pl.ds(k * C, C)], gsem)

    def xform(p):
      def tbody(k, carry):
        for j in range(C // 16):
          v = srcb[p, k, pl.ds(j * 16, 16)]
          srcb[p, k, pl.ds(j * 16, 16)] = v + v + c
        return carry
      lax.fori_loop(0, K, tbody, 0)

    d1, d2 = load_idx(0, 0)
    d1.wait()
    d2.wait()
    xform(0)
    fire(0)

    def gbody(g, carry):
      p = lax.rem(g, 2)
      q = 1 - p

      pltpu.make_async_copy(xtab.at[pl.ds(0, KC)], rows_v.at[p],
                            gsem).wait()
      for k in range(K):
        pltpu.async_copy(rows_v.at[p, pl.ds(k * C, C)],
                         agg_sh.at[dstb.at[p, k]], ssem, add=True)
        if first:
          pltpu.async_copy(ones_v, cnt_sh.at[dstb.at[p, k]], osem,
                           add=True)

      @pl.when(g > 0)
      def _():
        pltpu.make_async_copy(xtab.at[pl.ds(0, KC)], rows_v.at[q],
                              ssem).wait()
        if first:
          pltpu.make_async_copy(inv_out.at[pl.ds(0, KC)], drainb,
                                osem).wait()

      @pl.when(g + 1 < NG)
      def _():
        load_idx(g + 1, q)

      @pl.when(g + 1 < NG)
      def _():
        pltpu.make_async_copy(src2d.at[s, pl.ds(0, K)], srcb.at[q],
                              isem).wait()
        pltpu.make_async_copy(dst2d.at[s, pl.ds(0, K)], dstb.at[q],
                              isem).wait()
        xform(q)
        fire(q)
      return carry
    lax.fori_loop(0, NG, gbody, 0)

    pfin = lax.rem(NG - 1, 2)
    pltpu.make_async_copy(xtab.at[pl.ds(0, KC)], rows_v.at[pfin],
                          ssem).wait()
    if first:
      pltpu.make_async_copy(inv_out.at[pl.ds(0, KC)], drainb,
                            osem).wait()
    plsc.subcore_barrier()

    def sbody(r, carry):
      base = row0 + r * C
      pltpu.sync_copy(agg_sh.at[pl.ds(base, C)], rows_v.at[0, pl.ds(0, C)])
      if first:
        pltpu.sync_copy(cnt_sh.at[pl.ds(base, C)], cntb)
        for k in range(C // 16):
          ivb[pl.ds(k * 16, 16)] = 1.0 / jnp.maximum(
              cntb[pl.ds(k * 16, 16)], 1.0)

        @pl.when(c == 0)
        def _():
          pltpu.sync_copy(ivb.at[pl.ds(0, C)], inv_out.at[pl.ds(base, C)])
      else:
        pltpu.sync_copy(inv_in.at[pl.ds(base, C)], ivb.at[pl.ds(0, C)])

      def srow(i, carry2):
        iv = ivb[pl.ds(i, 16)][0]
        for k in range(H // 16):
          rows_v[0, i, pl.ds(k * 16, 16)] = rows_v[0, i, pl.ds(k * 16, 16)] * iv
        return carry2
      lax.fori_loop(0, C, srow, 0)
      pltpu.sync_copy(rows_v.at[0, pl.ds(0, C)],
                      mean_out.at[c, pl.ds(base, C)])
      return carry
    lax.fori_loop(0, RC, sbody, 0)

  return body


def _make_sc_kernel(first):
  if first:
    out_type = [jax.ShapeDtypeStruct((NC, NP, H), jnp.float32),
                jax.ShapeDtypeStruct((NP,), jnp.float32)]
  else:
    out_type = jax.ShapeDtypeStruct((NC, NP, H), jnp.float32)
  return pl.kernel(
      _make_sc_body(first),
      out_type=out_type,
      mesh=plsc.VectorSubcoreMesh(
          core_axis_name="c", subcore_axis_name="s",
          num_cores=NC, num_subcores=NS),
      scratch_types=[
          pltpu.VMEM((2, K, C), jnp.int32),
          pltpu.VMEM((2, K, C), jnp.int32),
          pltpu.VMEM((2, KC, H), jnp.float32),
          pltpu.VMEM((C,), jnp.float32),
          pltpu.VMEM((C,), jnp.float32),
          pltpu.VMEM((C + 16,), jnp.float32),
          pltpu.VMEM((KC,), jnp.float32),
          pltpu.VMEM_SHARED((NP, H), jnp.float32),
          pltpu.VMEM_SHARED((NP,), jnp.float32),
          pltpu.SemaphoreType.DMA,
          pltpu.SemaphoreType.DMA,
          pltpu.SemaphoreType.DMA,
          pltpu.SemaphoreType.DMA,
      ],
      compiler_params=pltpu.CompilerParams(use_tc_tiling_on_sc=False),
  )


_sc_agg_first = _make_sc_kernel(True)
_sc_agg_next = _make_sc_kernel(False)


def _layer_norm_silu(h, g, be):
  mu = jnp.mean(h, axis=-1, keepdims=True)
  d = h - mu
  var = jnp.mean(d * d, axis=-1, keepdims=True)
  hn = d * lax.rsqrt(var + 1e-5) * g + be
  return hn * jax.nn.sigmoid(hn)


def _stage1_body(meanp_ref, x_ref, Wl_ref, Wr_ref, b_ref, g_ref, be_ref,
                 out_ref):
  mean = jnp.concatenate([meanp_ref[0], meanp_ref[1]], axis=-1)
  h = (jnp.dot(mean, Wl_ref[...], preferred_element_type=jnp.float32)
       + jnp.dot(x_ref[...], Wr_ref[...], preferred_element_type=jnp.float32)
       + b_ref[...])
  sil = _layer_norm_silu(h, g_ref[...], be_ref[...])
  out_ref[:, 0, :] = sil[:, :H]
  out_ref[:, 1, :] = sil[:, H:]


def _stage2_body(meanp_ref, hp_ref, Wl_ref, Wr_ref, b_ref, g_ref, be_ref,
                 Wm1_ref, bm1_ref, Wm2_ref, bm2_ref, out_ref):
  mean = jnp.concatenate([meanp_ref[0], meanp_ref[1]], axis=-1)
  hprev = jnp.concatenate([hp_ref[:, 0, :], hp_ref[:, 1, :]], axis=-1)
  h = (jnp.dot(mean, Wl_ref[...], preferred_element_type=jnp.float32)
       + jnp.dot(hprev, Wr_ref[...], preferred_element_type=jnp.float32)
       + b_ref[...])
  sil = _layer_norm_silu(h, g_ref[...], be_ref[...])
  m = jnp.maximum(
      jnp.dot(sil, Wm1_ref[...], preferred_element_type=jnp.float32)
      + bm1_ref[...], 0.0)
  out_ref[...] = (jnp.dot(m, Wm2_ref[...], preferred_element_type=jnp.float32)
                  + bm2_ref[...])


_half_spec = pl.BlockSpec((NC, BT, H), lambda i: (0, i, 0))
_full_spec = pl.BlockSpec((BT, D), lambda i: (i, 0))
_w_spec = pl.BlockSpec((D, D), lambda i: (0, 0))
_v_spec = pl.BlockSpec((1, D), lambda i: (0, 0))
_w2_spec = pl.BlockSpec((D, 1), lambda i: (0, 0))
_v2_spec = pl.BlockSpec((1, 1), lambda i: (0, 0))
_hp_spec = pl.BlockSpec((BT, 2, H), lambda i: (i, 0, 0))

_stage1 = pl.pallas_call(
    _stage1_body,
    grid=(G,),
    in_specs=[_half_spec, _full_spec, _w_spec, _w_spec,
              _v_spec, _v_spec, _v_spec],
    out_specs=_hp_spec,
    out_shape=jax.ShapeDtypeStruct((N, 2, H), jnp.float32),
)

_stage2 = pl.pallas_call(
    _stage2_body,
    grid=(G,),
    in_specs=[_half_spec, _hp_spec, _w_spec, _w_spec,
              _v_spec, _v_spec, _v_spec,
              _w_spec, _v_spec, _w2_spec, _v2_spec],
    out_specs=pl.BlockSpec((BT, 1), lambda i: (i, 0)),
    out_shape=jax.ShapeDtypeStruct((N, 1), jnp.float32),
)


@jax.jit
def kernel(x, edge_index, Wl0, Wr0, b0, g0, be0, Wl1, Wr1, b1, g1, be1,
           Wm1, bm1, Wm2, bm2):
  ei = edge_index.astype(jnp.int32)
  src2d = ei[0].reshape(NS, NCHUNK, C)
  dst2d = ei[1].reshape(NS, NCHUNK, C)

  meanp0, inv = _sc_agg_first(x.reshape(2 * N, H), src2d, dst2d)
  hp = _stage1(meanp0, x, Wl0, Wr0,
               b0.reshape(1, D), g0.reshape(1, D), be0.reshape(1, D))
  meanp1 = _sc_agg_next(hp.reshape(2 * N, H), src2d, dst2d, inv)
  return _stage2(meanp1, hp, Wl1, Wr1,
                 b1.reshape(1, D), g1.reshape(1, D), be1.reshape(1, D),
                 Wm1, bm1.reshape(1, D), Wm2, bm2.reshape(1, 1))


# speedup vs baseline: 1.3368x; 1.2542x over previous
import functools

import jax
import jax.numpy as jnp
from jax import lax
from jax.experimental import pallas as pl
from jax.experimental.pallas import tpu as pltpu
from jax.experimental.pallas import tpu_sc as plsc

N = 10000
E = 320000
D = 128
H = 64
NC = 2
NS = 16
NP = 10240
BT = 1000
G = N // BT
EP = E // NS
C = 80
NCHUNK = EP // C
RP = NP // NS


K = 5
KC = K * C
NG = NCHUNK // K
RC = RP // C


def _make_sc_body(first):
  def body(xtab, src2d, dst2d, *rest):
    if first:
      mean_out, inv_out, srcb, dstb, rows_v, ones_v, cntb, ivb, drainb, \
          agg_sh, cnt_sh, gsem, ssem, osem, isem = rest
    else:
      inv_in, mean_out, srcb, dstb, rows_v, ones_v, cntb, ivb, drainb, \
          agg_sh, cnt_sh, gsem, ssem, osem, isem = rest
    c = lax.axis_index("c")
    s = lax.axis_index("s")
    row0 = s * RP

    z16 = jnp.zeros((16,), jnp.float32)
    o16 = jnp.ones((16,), jnp.float32)

    def zrow(i, carry):
      for k in range(H // 16):
        rows_v[0, i, pl.ds(k * 16, 16)] = z16
      return carry
    lax.fori_loop(0, C, zrow, 0)
    for k in range(C // 16):
      cntb[pl.ds(k * 16, 16)] = z16
      ones_v[pl.ds(k * 16, 16)] = o16
    for r in range(RC):
      pltpu.sync_copy(rows_v.at[0, pl.ds(0, C)],
                      agg_sh.at[pl.ds(row0 + r * C, C)])
      if first:
        pltpu.sync_copy(cntb, cnt_sh.at[pl.ds(row0 + r * C, C)])

    plsc.subcore_barrier()

    def load_idx(g, p):
      d1 = pltpu.async_copy(src2d.at[s, pl.ds(g * K, K)], srcb.at[p], isem)
      d2 = pltpu.async_copy(dst2d.at[s, pl.ds(g * K, K)], dstb.at[p], isem)
      return d1, d2

    def fire(rs, isl):
      for k in range(K):
        pltpu.async_copy(xtab.at[srcb.at[isl, k]],
                         rows_v.at[rs, pl.ds(k * C, C)], gsem)

    def xform(p):
      def tbody(k, carry):
        for j in range(C // 16):
          v = srcb[p, k, pl.ds(j * 16, 16)]
          srcb[p, k, pl.ds(j * 16, 16)] = v + v + c
        return carry
      lax.fori_loop(0, K, tbody, 0)

    d1, d2 = load_idx(0, 0)
    d1.wait()
    d2.wait()
    xform(0)
    d3, d4 = load_idx(1, 1)
    fire(0, 0)
    d3.wait()
    d4.wait()
    xform(1)
    fire(1, 1)

    def gbody(g, carry):
      r = lax.rem(g, 3)
      rn = lax.rem(g + 2, 3)
      ic = lax.rem(g, 4)
      inx = lax.rem(g + 2, 4)

      @pl.when(g + 2 < NG)
      def _():
        load_idx(g + 2, inx)

      pltpu.make_async_copy(xtab.at[pl.ds(0, KC)], rows_v.at[r],
                            gsem).wait()
      for k in range(K):
        pltpu.async_copy(rows_v.at[r, pl.ds(k * C, C)],
                         agg_sh.at[dstb.at[ic, k]], ssem, add=True)
        if first:
          pltpu.async_copy(ones_v, cnt_sh.at[dstb.at[ic, k]], osem,
                           add=True)

      @pl.when(g > 0)
      def _():
        pltpu.make_async_copy(xtab.at[pl.ds(0, KC)], rows_v.at[rn],
                              ssem).wait()
        if first:
          pltpu.make_async_copy(inv_out.at[pl.ds(0, KC)], drainb,
                                osem).wait()

      @pl.when(g + 2 < NG)
      def _():
        pltpu.make_async_copy(src2d.at[s, pl.ds(0, K)], srcb.at[inx],
                              isem).wait()
        pltpu.make_async_copy(dst2d.at[s, pl.ds(0, K)], dstb.at[inx],
                              isem).wait()
        xform(inx)
        fire(rn, inx)
      return carry
    lax.fori_loop(0, NG, gbody, 0)

    pltpu.make_async_copy(xtab.at[pl.ds(0, KC)],
                          rows_v.at[lax.rem(NG - 1, 3)], ssem).wait()
    if first:
      pltpu.make_async_copy(inv_out.at[pl.ds(0, KC)], drainb,
                            osem).wait()
    plsc.subcore_barrier()

    def sbody(r, carry):
      base = row0 + r * C
      pltpu.sync_copy(agg_sh.at[pl.ds(base, C)], rows_v.at[0, pl.ds(0, C)])
      if first:
        pltpu.sync_copy(cnt_sh.at[pl.ds(base, C)], cntb)
        for k in range(C // 16):
          ivb[pl.ds(k * 16, 16)] = 1.0 / jnp.maximum(
              cntb[pl.ds(k * 16, 16)], 1.0)

        @pl.when(c == 0)
        def _():
          pltpu.sync_copy(ivb.at[pl.ds(0, C)], inv_out.at[pl.ds(base, C)])
      else:
        pltpu.sync_copy(inv_in.at[pl.ds(base, C)], ivb.at[pl.ds(0, C)])

      def srow(i, carry2):
        iv = ivb[pl.ds(i, 16)][0]
        for k in range(H // 16):
          rows_v[0, i, pl.ds(k * 16, 16)] = rows_v[0, i, pl.ds(k * 16, 16)] * iv
        return carry2
      lax.fori_loop(0, C, srow, 0)
      pltpu.sync_copy(rows_v.at[0, pl.ds(0, C)],
                      mean_out.at[c, pl.ds(base, C)])
      return carry
    lax.fori_loop(0, RC, sbody, 0)

  return body


def _make_sc_kernel(first):
  if first:
    out_type = [jax.ShapeDtypeStruct((NC, NP, H), jnp.float32),
                jax.ShapeDtypeStruct((NP,), jnp.float32)]
  else:
    out_type = jax.ShapeDtypeStruct((NC, NP, H), jnp.float32)
  return pl.kernel(
      _make_sc_body(first),
      out_type=out_type,
      mesh=plsc.VectorSubcoreMesh(
          core_axis_name="c", subcore_axis_name="s",
          num_cores=NC, num_subcores=NS),
      scratch_types=[
          pltpu.VMEM((4, K, C), jnp.int32),
          pltpu.VMEM((4, K, C), jnp.int32),
          pltpu.VMEM((3, KC, H), jnp.float32),
          pltpu.VMEM((C,), jnp.float32),
          pltpu.VMEM((C,), jnp.float32),
          pltpu.VMEM((C + 16,), jnp.float32),
          pltpu.VMEM((KC,), jnp.float32),
          pltpu.VMEM_SHARED((NP, H), jnp.float32),
          pltpu.VMEM_SHARED((NP,), jnp.float32),
          pltpu.SemaphoreType.DMA,
          pltpu.SemaphoreType.DMA,
          pltpu.SemaphoreType.DMA,
          pltpu.SemaphoreType.DMA,
      ],
      compiler_params=pltpu.CompilerParams(use_tc_tiling_on_sc=False),
  )


_sc_agg_first = _make_sc_kernel(True)
_sc_agg_next = _make_sc_kernel(False)


def _layer_norm_silu(h, g, be):
  mu = jnp.mean(h, axis=-1, keepdims=True)
  d = h - mu
  var = jnp.mean(d * d, axis=-1, keepdims=True)
  hn = d * lax.rsqrt(var + 1e-5) * g + be
  return hn * jax.nn.sigmoid(hn)


def _stage1_body(meanp_ref, x_ref, Wl_ref, Wr_ref, b_ref, g_ref, be_ref,
                 out_ref):
  mean = jnp.concatenate([meanp_ref[0], meanp_ref[1]], axis=-1)
  h = (jnp.dot(mean, Wl_ref[...], preferred_element_type=jnp.float32)
       + jnp.dot(x_ref[...], Wr_ref[...], preferred_element_type=jnp.float32)
       + b_ref[...])
  sil = _layer_norm_silu(h, g_ref[...], be_ref[...])
  out_ref[:, 0, :] = sil[:, :H]
  out_ref[:, 1, :] = sil[:, H:]


def _stage2_body(meanp_ref, hp_ref, Wl_ref, Wr_ref, b_ref, g_ref, be_ref,
                 Wm1_ref, bm1_ref, Wm2_ref, bm2_ref, out_ref):
  mean = jnp.concatenate([meanp_ref[0], meanp_ref[1]], axis=-1)
  hprev = jnp.concatenate([hp_ref[:, 0, :], hp_ref[:, 1, :]], axis=-1)
  h = (jnp.dot(mean, Wl_ref[...], preferred_element_type=jnp.float32)
       + jnp.dot(hprev, Wr_ref[...], preferred_element_type=jnp.float32)
       + b_ref[...])
  sil = _layer_norm_silu(h, g_ref[...], be_ref[...])
  m = jnp.maximum(
      jnp.dot(sil, Wm1_ref[...], preferred_element_type=jnp.float32)
      + bm1_ref[...], 0.0)
  out_ref[...] = (jnp.dot(m, Wm2_ref[...], preferred_element_type=jnp.float32)
                  + bm2_ref[...])


_half_spec = pl.BlockSpec((NC, BT, H), lambda i: (0, i, 0))
_full_spec = pl.BlockSpec((BT, D), lambda i: (i, 0))
_w_spec = pl.BlockSpec((D, D), lambda i: (0, 0))
_v_spec = pl.BlockSpec((1, D), lambda i: (0, 0))
_w2_spec = pl.BlockSpec((D, 1), lambda i: (0, 0))
_v2_spec = pl.BlockSpec((1, 1), lambda i: (0, 0))
_hp_spec = pl.BlockSpec((BT, 2, H), lambda i: (i, 0, 0))

_stage1 = pl.pallas_call(
    _stage1_body,
    grid=(G,),
    in_specs=[_half_spec, _full_spec, _w_spec, _w_spec,
              _v_spec, _v_spec, _v_spec],
    out_specs=_hp_spec,
    out_shape=jax.ShapeDtypeStruct((N, 2, H), jnp.float32),
)

_stage2 = pl.pallas_call(
    _stage2_body,
    grid=(G,),
    in_specs=[_half_spec, _hp_spec, _w_spec, _w_spec,
              _v_spec, _v_spec, _v_spec,
              _w_spec, _v_spec, _w2_spec, _v2_spec],
    out_specs=pl.BlockSpec((BT, 1), lambda i: (i, 0)),
    out_shape=jax.ShapeDtypeStruct((N, 1), jnp.float32),
)


@jax.jit
def kernel(x, edge_index, Wl0, Wr0, b0, g0, be0, Wl1, Wr1, b1, g1, be1,
           Wm1, bm1, Wm2, bm2):
  ei = edge_index.astype(jnp.int32)
  src2d = ei[0].reshape(NS, NCHUNK, C)
  dst2d = ei[1].reshape(NS, NCHUNK, C)

  meanp0, inv = _sc_agg_first(x.reshape(2 * N, H), src2d, dst2d)
  hp = _stage1(meanp0, x, Wl0, Wr0,
               b0.reshape(1, D), g0.reshape(1, D), be0.reshape(1, D))
  meanp1 = _sc_agg_next(hp.reshape(2 * N, H), src2d, dst2d, inv)
  return _stage2(meanp1, hp, Wl1, Wr1,
                 b1.reshape(1, D), g1.reshape(1, D), be1.reshape(1, D),
                 Wm1, bm1.reshape(1, D), Wm2, bm2.reshape(1, 1))


# speedup vs baseline: 1.3769x; 1.0300x over previous
import functools

import jax
import jax.numpy as jnp
from jax import lax
from jax.experimental import pallas as pl
from jax.experimental.pallas import tpu as pltpu
from jax.experimental.pallas import tpu_sc as plsc

N = 10000
E = 320000
D = 128
H = 64
NC = 2
NS = 16
NP = 10240
BT = 1000
G = N // BT
EP = E // NS
C = 80
NCHUNK = EP // C
RP = NP // NS


K = 5
KC = K * C
NG = NCHUNK // K
RC = RP // C


def _make_sc_body(first):
  def body(xtab, src2d, dst2d, *rest):
    if first:
      mean_out, inv_out, srcb, dstb, rows_v, ones_v, cntb, ivb, drainb, \
          agg_sh, cnt_sh, gsem, ssem, osem, isem = rest
    else:
      inv_in, mean_out, srcb, dstb, rows_v, ones_v, cntb, ivb, drainb, \
          agg_sh, cnt_sh, gsem, ssem, osem, isem = rest
    c = lax.axis_index("c")
    s = lax.axis_index("s")
    row0 = s * RP

    z16 = jnp.zeros((16,), jnp.float32)
    o16 = jnp.ones((16,), jnp.float32)

    def zrow(i, carry):
      for k in range(H // 16):
        rows_v[0, i, pl.ds(k * 16, 16)] = z16
      return carry
    lax.fori_loop(0, C, zrow, 0)
    for k in range(C // 16):
      cntb[0, pl.ds(k * 16, 16)] = z16
      ones_v[pl.ds(k * 16, 16)] = o16
    for r in range(RC):
      pltpu.async_copy(rows_v.at[0, pl.ds(0, C)],
                       agg_sh.at[pl.ds(row0 + r * C, C)], gsem)
      if first:
        pltpu.async_copy(cntb.at[0], cnt_sh.at[pl.ds(row0 + r * C, C)],
                         osem)
    pltpu.make_async_copy(xtab.at[pl.ds(0, KC)], rows_v.at[0], gsem).wait()
    pltpu.make_async_copy(xtab.at[pl.ds(0, RC * C - KC)],
                          rows_v.at[0, pl.ds(0, RC * C - KC)], gsem).wait()
    if first:
      for r in range(RC):
        pltpu.make_async_copy(inv_out.at[pl.ds(0, C)], cntb.at[0],
                              osem).wait()

    plsc.subcore_barrier()

    def load_idx(g, p):
      d1 = pltpu.async_copy(src2d.at[s, pl.ds(g * K, K)], srcb.at[p], isem)
      d2 = pltpu.async_copy(dst2d.at[s, pl.ds(g * K, K)], dstb.at[p], isem)
      return d1, d2

    def fire(rs, isl):
      for k in range(K):
        pltpu.async_copy(xtab.at[srcb.at[isl, k]],
                         rows_v.at[rs, pl.ds(k * C, C)], gsem)

    def xform(p):
      def tbody(k, carry):
        for j in range(C // 16):
          v = srcb[p, k, pl.ds(j * 16, 16)]
          srcb[p, k, pl.ds(j * 16, 16)] = v + v + c
        return carry
      lax.fori_loop(0, K, tbody, 0)

    d1, d2 = load_idx(0, 0)
    d1.wait()
    d2.wait()
    xform(0)
    d3, d4 = load_idx(1, 1)
    fire(0, 0)
    d3.wait()
    d4.wait()
    xform(1)
    fire(1, 1)

    def gbody(g, carry):
      r = lax.rem(g, 3)
      rn = lax.rem(g + 2, 3)
      ic = lax.rem(g, 4)
      inx = lax.rem(g + 2, 4)

      @pl.when(g + 2 < NG)
      def _():
        load_idx(g + 2, inx)

      pltpu.make_async_copy(xtab.at[pl.ds(0, KC)], rows_v.at[r],
                            gsem).wait()
      for k in range(K):
        pltpu.async_copy(rows_v.at[r, pl.ds(k * C, C)],
                         agg_sh.at[dstb.at[ic, k]], ssem, add=True)
        if first:
          pltpu.async_copy(ones_v, cnt_sh.at[dstb.at[ic, k]], osem,
                           add=True)

      @pl.when(g > 0)
      def _():
        pltpu.make_async_copy(xtab.at[pl.ds(0, KC)], rows_v.at[rn],
                              ssem).wait()
        if first:
          pltpu.make_async_copy(inv_out.at[pl.ds(0, KC)], drainb,
                                osem).wait()

      @pl.when(g + 2 < NG)
      def _():
        pltpu.make_async_copy(src2d.at[s, pl.ds(0, K)], srcb.at[inx],
                              isem).wait()
        pltpu.make_async_copy(dst2d.at[s, pl.ds(0, K)], dstb.at[inx],
                              isem).wait()
        xform(inx)
        fire(rn, inx)
      return carry
    lax.fori_loop(0, NG, gbody, 0)

    pltpu.make_async_copy(xtab.at[pl.ds(0, KC)],
                          rows_v.at[lax.rem(NG - 1, 3)], ssem).wait()
    if first:
      pltpu.make_async_copy(inv_out.at[pl.ds(0, KC)], drainb,
                            osem).wait()
    plsc.subcore_barrier()

    def issue_in(r, p):
      pltpu.async_copy(agg_sh.at[pl.ds(row0 + r * C, C)],
                       rows_v.at[p, pl.ds(0, C)], gsem)
      if first:
        pltpu.async_copy(cnt_sh.at[pl.ds(row0 + r * C, C)], cntb.at[p],
                         gsem)
      else:
        pltpu.async_copy(inv_in.at[pl.ds(row0 + r * C, C)],
                         ivb.at[p, pl.ds(0, C)], isem)

    issue_in(0, 0)

    def sbody(r, carry):
      p = lax.rem(r, 2)
      q = 1 - p
      base = row0 + r * C
      pltpu.make_async_copy(xtab.at[pl.ds(0, C)],
                            rows_v.at[p, pl.ds(0, C)], gsem).wait()
      if first:
        pltpu.make_async_copy(inv_out.at[pl.ds(0, C)], cntb.at[p],
                              gsem).wait()
      else:
        pltpu.make_async_copy(inv_in.at[pl.ds(0, C)],
                              ivb.at[p, pl.ds(0, C)], isem).wait()

      @pl.when(r > 0)
      def _():
        pltpu.make_async_copy(xtab.at[pl.ds(0, C)],
                              rows_v.at[q, pl.ds(0, C)], ssem).wait()

      @pl.when(r + 1 < RC)
      def _():
        issue_in(r + 1, q)

      if first:
        for k in range(C // 16):
          ivb[p, pl.ds(k * 16, 16)] = 1.0 / jnp.maximum(
              cntb[p, pl.ds(k * 16, 16)], 1.0)

        @pl.when(c == 0)
        def _():
          pltpu.sync_copy(ivb.at[p, pl.ds(0, C)],
                          inv_out.at[pl.ds(base, C)])

      def srow(i, carry2):
        iv = ivb[p, pl.ds(i, 16)][0]
        for k in range(H // 16):
          rows_v[p, i, pl.ds(k * 16, 16)] = (
              rows_v[p, i, pl.ds(k * 16, 16)] * iv)
        return carry2
      lax.fori_loop(0, C, srow, 0)
      pltpu.async_copy(rows_v.at[p, pl.ds(0, C)],
                       mean_out.at[c, pl.ds(base, C)], ssem)
      return carry
    lax.fori_loop(0, RC, sbody, 0)

    pltpu.make_async_copy(xtab.at[pl.ds(0, C)],
                          rows_v.at[lax.rem(RC - 1, 2), pl.ds(0, C)],
                          ssem).wait()

  return body


def _make_sc_kernel(first):
  if first:
    out_type = [jax.ShapeDtypeStruct((NC, NP, H), jnp.float32),
                jax.ShapeDtypeStruct((NP,), jnp.float32)]
  else:
    out_type = jax.ShapeDtypeStruct((NC, NP, H), jnp.float32)
  return pl.kernel(
      _make_sc_body(first),
      out_type=out_type,
      mesh=plsc.VectorSubcoreMesh(
          core_axis_name="c", subcore_axis_name="s",
          num_cores=NC, num_subcores=NS),
      scratch_types=[
          pltpu.VMEM((4, K, C), jnp.int32),
          pltpu.VMEM((4, K, C), jnp.int32),
          pltpu.VMEM((3, KC, H), jnp.float32),
          pltpu.VMEM((C,), jnp.float32),
          pltpu.VMEM((2, C), jnp.float32),
          pltpu.VMEM((2, C + 16), jnp.float32),
          pltpu.VMEM((KC,), jnp.float32),
          pltpu.VMEM_SHARED((NP, H), jnp.float32),
          pltpu.VMEM_SHARED((NP,), jnp.float32),
          pltpu.SemaphoreType.DMA,
          pltpu.SemaphoreType.DMA,
          pltpu.SemaphoreType.DMA,
          pltpu.SemaphoreType.DMA,
      ],
      compiler_params=pltpu.CompilerParams(use_tc_tiling_on_sc=False),
  )


_sc_agg_first = _make_sc_kernel(True)
_sc_agg_next = _make_sc_kernel(False)


def _layer_norm_silu(h, g, be):
  mu = jnp.mean(h, axis=-1, keepdims=True)
  d = h - mu
  var = jnp.mean(d * d, axis=-1, keepdims=True)
  hn = d * lax.rsqrt(var + 1e-5) * g + be
  return hn * jax.nn.sigmoid(hn)


def _stage1_body(meanp_ref, x_ref, Wl_ref, Wr_ref, b_ref, g_ref, be_ref,
                 out_ref):
  mean = jnp.concatenate([meanp_ref[0], meanp_ref[1]], axis=-1)
  h = (jnp.dot(mean, Wl_ref[...], preferred_element_type=jnp.float32)
       + jnp.dot(x_ref[...], Wr_ref[...], preferred_element_type=jnp.float32)
       + b_ref[...])
  sil = _layer_norm_silu(h, g_ref[...], be_ref[...])
  out_ref[:, 0, :] = sil[:, :H]
  out_ref[:, 1, :] = sil[:, H:]


def _stage2_body(meanp_ref, hp_ref, Wl_ref, Wr_ref, b_ref, g_ref, be_ref,
                 Wm1_ref, bm1_ref, Wm2_ref, bm2_ref, out_ref):
  mean = jnp.concatenate([meanp_ref[0], meanp_ref[1]], axis=-1)
  hprev = jnp.concatenate([hp_ref[:, 0, :], hp_ref[:, 1, :]], axis=-1)
  h = (jnp.dot(mean, Wl_ref[...], preferred_element_type=jnp.float32)
       + jnp.dot(hprev, Wr_ref[...], preferred_element_type=jnp.float32)
       + b_ref[...])
  sil = _layer_norm_silu(h, g_ref[...], be_ref[...])
  m = jnp.maximum(
      jnp.dot(sil, Wm1_ref[...], preferred_element_type=jnp.float32)
      + bm1_ref[...], 0.0)
  out_ref[...] = (jnp.dot(m, Wm2_ref[...], preferred_element_type=jnp.float32)
                  + bm2_ref[...])


_half_spec = pl.BlockSpec((NC, BT, H), lambda i: (0, i, 0))
_full_spec = pl.BlockSpec((BT, D), lambda i: (i, 0))
_w_spec = pl.BlockSpec((D, D), lambda i: (0, 0))
_v_spec = pl.BlockSpec((1, D), lambda i: (0, 0))
_w2_spec = pl.BlockSpec((D, 1), lambda i: (0, 0))
_v2_spec = pl.BlockSpec((1, 1), lambda i: (0, 0))
_hp_spec = pl.BlockSpec((BT, 2, H), lambda i: (i, 0, 0))

_stage1 = pl.pallas_call(
    _stage1_body,
    grid=(G,),
    in_specs=[_half_spec, _full_spec, _w_spec, _w_spec,
              _v_spec, _v_spec, _v_spec],
    out_specs=_hp_spec,
    out_shape=jax.ShapeDtypeStruct((N, 2, H), jnp.float32),
)

_stage2 = pl.pallas_call(
    _stage2_body,
    grid=(G,),
    in_specs=[_half_spec, _hp_spec, _w_spec, _w_spec,
              _v_spec, _v_spec, _v_spec,
              _w_spec, _v_spec, _w2_spec, _v2_spec],
    out_specs=pl.BlockSpec((BT, 1), lambda i: (i, 0)),
    out_shape=jax.ShapeDtypeStruct((N, 1), jnp.float32),
)


@jax.jit
def kernel(x, edge_index, Wl0, Wr0, b0, g0, be0, Wl1, Wr1, b1, g1, be1,
           Wm1, bm1, Wm2, bm2):
  ei = edge_index.astype(jnp.int32)
  src2d = ei[0].reshape(NS, NCHUNK, C)
  dst2d = ei[1].reshape(NS, NCHUNK, C)

  meanp0, inv = _sc_agg_first(x.reshape(2 * N, H), src2d, dst2d)
  hp = _stage1(meanp0, x, Wl0, Wr0,
               b0.reshape(1, D), g0.reshape(1, D), be0.reshape(1, D))
  meanp1 = _sc_agg_next(hp.reshape(2 * N, H), src2d, dst2d, inv)
  return _stage2(meanp1, hp, Wl1, Wr1,
                 b1.reshape(1, D), g1.reshape(1, D), be1.reshape(1, D),
                 Wm1, bm1.reshape(1, D), Wm2, bm2.reshape(1, 1))


# speedup vs baseline: 1.4137x; 1.0267x over previous
import functools

import jax
import jax.numpy as jnp
from jax import lax
from jax.experimental import pallas as pl
from jax.experimental.pallas import tpu as pltpu
from jax.experimental.pallas import tpu_sc as plsc

N = 10000
E = 320000
D = 128
H = 64
NC = 2
NS = 16
NP = 10240
BT = 2000
G = N // BT
EP = E // NS
C = 80
NCHUNK = EP // C
RP = NP // NS


K = 5
KC = K * C
NG = NCHUNK // K
RC = RP // C


def _make_sc_body(first):
  def body(xtab, src2d, dst2d, *rest):
    if first:
      mean_out, inv_out, srcb, dstb, rows_v, ones_v, cntb, ivb, drainb, \
          agg_sh, cnt_sh, gsem, ssem, osem, isem = rest
    else:
      inv_in, mean_out, srcb, dstb, rows_v, ones_v, cntb, ivb, drainb, \
          agg_sh, cnt_sh, gsem, ssem, osem, isem = rest
    c = lax.axis_index("c")
    s = lax.axis_index("s")
    row0 = s * RP

    z16 = jnp.zeros((16,), jnp.float32)
    o16 = jnp.ones((16,), jnp.float32)

    def zrow(i, carry):
      for k in range(H // 16):
        rows_v[0, i, pl.ds(k * 16, 16)] = z16
      return carry
    lax.fori_loop(0, C, zrow, 0)
    for k in range(C // 16):
      cntb[0, pl.ds(k * 16, 16)] = z16
      ones_v[pl.ds(k * 16, 16)] = o16
    for r in range(RC):
      pltpu.async_copy(rows_v.at[0, pl.ds(0, C)],
                       agg_sh.at[pl.ds(row0 + r * C, C)], gsem)
      if first:
        pltpu.async_copy(cntb.at[0], cnt_sh.at[pl.ds(row0 + r * C, C)],
                         osem)
    pltpu.make_async_copy(xtab.at[pl.ds(0, KC)], rows_v.at[0], gsem).wait()
    pltpu.make_async_copy(xtab.at[pl.ds(0, RC * C - KC)],
                          rows_v.at[0, pl.ds(0, RC * C - KC)], gsem).wait()
    if first:
      for r in range(RC):
        pltpu.make_async_copy(inv_out.at[pl.ds(0, C)], cntb.at[0],
                              osem).wait()

    plsc.subcore_barrier()

    def load_idx(g, p):
      d1 = pltpu.async_copy(src2d.at[s, pl.ds(g * K, K)], srcb.at[p], isem)
      d2 = pltpu.async_copy(dst2d.at[s, pl.ds(g * K, K)], dstb.at[p], isem)
      return d1, d2

    def fire(rs, isl):
      for k in range(K):
        pltpu.async_copy(xtab.at[srcb.at[isl, k]],
                         rows_v.at[rs, pl.ds(k * C, C)], gsem)

    def xform(p):
      def tbody(k, carry):
        for j in range(C // 16):
          v = srcb[p, k, pl.ds(j * 16, 16)]
          srcb[p, k, pl.ds(j * 16, 16)] = v + v + c
        return carry
      lax.fori_loop(0, K, tbody, 0)

    d1, d2 = load_idx(0, 0)
    d1.wait()
    d2.wait()
    xform(0)
    d3, d4 = load_idx(1, 1)
    fire(0, 0)
    d3.wait()
    d4.wait()
    xform(1)
    fire(1, 1)

    def gbody(g, carry):
      r = lax.rem(g, 3)
      rn = lax.rem(g + 2, 3)
      ic = lax.rem(g, 4)
      inx = lax.rem(g + 2, 4)

      @pl.when(g + 2 < NG)
      def _():
        load_idx(g + 2, inx)

      pltpu.make_async_copy(xtab.at[pl.ds(0, KC)], rows_v.at[r],
                            gsem).wait()
      for k in range(K):
        pltpu.async_copy(rows_v.at[r, pl.ds(k * C, C)],
                         agg_sh.at[dstb.at[ic, k]], ssem, add=True)
        if first:
          pltpu.async_copy(ones_v, cnt_sh.at[dstb.at[ic, k]], osem,
                           add=True)

      @pl.when(g > 0)
      def _():
        pltpu.make_async_copy(xtab.at[pl.ds(0, KC)], rows_v.at[rn],
                              ssem).wait()
        if first:
          pltpu.make_async_copy(inv_out.at[pl.ds(0, KC)], drainb,
                                osem).wait()

      @pl.when(g + 2 < NG)
      def _():
        pltpu.make_async_copy(src2d.at[s, pl.ds(0, K)], srcb.at[inx],
                              isem).wait()
        pltpu.make_async_copy(dst2d.at[s, pl.ds(0, K)], dstb.at[inx],
                              isem).wait()
        xform(inx)
        fire(rn, inx)
      return carry
    lax.fori_loop(0, NG, gbody, 0)

    pltpu.make_async_copy(xtab.at[pl.ds(0, KC)],
                          rows_v.at[lax.rem(NG - 1, 3)], ssem).wait()
    if first:
      pltpu.make_async_copy(inv_out.at[pl.ds(0, KC)], drainb,
                            osem).wait()
    plsc.subcore_barrier()

    def issue_in(r, p):
      pltpu.async_copy(agg_sh.at[pl.ds(row0 + r * C, C)],
                       rows_v.at[p, pl.ds(0, C)], gsem)
      if first:
        pltpu.async_copy(cnt_sh.at[pl.ds(row0 + r * C, C)], cntb.at[p],
                         gsem)
      else:
        pltpu.async_copy(inv_in.at[pl.ds(row0 + r * C, C)],
                         ivb.at[p, pl.ds(0, C)], isem)

    issue_in(0, 0)

    def sbody(r, carry):
      p = lax.rem(r, 2)
      q = 1 - p
      base = row0 + r * C
      pltpu.make_async_copy(xtab.at[pl.ds(0, C)],
                            rows_v.at[p, pl.ds(0, C)], gsem).wait()
      if first:
        pltpu.make_async_copy(inv_out.at[pl.ds(0, C)], cntb.at[p],
                              gsem).wait()
      else:
        pltpu.make_async_copy(inv_in.at[pl.ds(0, C)],
                              ivb.at[p, pl.ds(0, C)], isem).wait()

      @pl.when(r > 0)
      def _():
        pltpu.make_async_copy(xtab.at[pl.ds(0, C)],
                              rows_v.at[q, pl.ds(0, C)], ssem).wait()

      @pl.when(r + 1 < RC)
      def _():
        issue_in(r + 1, q)

      if first:
        for k in range(C // 16):
          ivb[p, pl.ds(k * 16, 16)] = 1.0 / jnp.maximum(
              cntb[p, pl.ds(k * 16, 16)], 1.0)

        @pl.when(c == 0)
        def _():
          pltpu.sync_copy(ivb.at[p, pl.ds(0, C)],
                          inv_out.at[pl.ds(base, C)])

      def srow(i, carry2):
        iv = ivb[p, pl.ds(i, 16)][0]
        for k in range(H // 16):
          rows_v[p, i, pl.ds(k * 16, 16)] = (
              rows_v[p, i, pl.ds(k * 16, 16)] * iv)
        return carry2
      lax.fori_loop(0, C, srow, 0)
      pltpu.async_copy(rows_v.at[p, pl.ds(0, C)],
                       mean_out.at[c, pl.ds(base, C)], ssem)
      return carry
    lax.fori_loop(0, RC, sbody, 0)

    pltpu.make_async_copy(xtab.at[pl.ds(0, C)],
                          rows_v.at[lax.rem(RC - 1, 2), pl.ds(0, C)],
                          ssem).wait()

  return body


def _make_sc_kernel(first):
  if first:
    out_type = [jax.ShapeDtypeStruct((NC, NP, H), jnp.float32),
                jax.ShapeDtypeStruct((NP,), jnp.float32)]
  else:
    out_type = jax.ShapeDtypeStruct((NC, NP, H), jnp.float32)
  return pl.kernel(
      _make_sc_body(first),
      out_type=out_type,
      mesh=plsc.VectorSubcoreMesh(
          core_axis_name="c", subcore_axis_name="s",
          num_cores=NC, num_subcores=NS),
      scratch_types=[
          pltpu.VMEM((4, K, C), jnp.int32),
          pltpu.VMEM((4, K, C), jnp.int32),
          pltpu.VMEM((3, KC, H), jnp.float32),
          pltpu.VMEM((C,), jnp.float32),
          pltpu.VMEM((2, C), jnp.float32),
          pltpu.VMEM((2, C + 16), jnp.float32),
          pltpu.VMEM((KC,), jnp.float32),
          pltpu.VMEM_SHARED((NP, H), jnp.float32),
          pltpu.VMEM_SHARED((NP,), jnp.float32),
          pltpu.SemaphoreType.DMA,
          pltpu.SemaphoreType.DMA,
          pltpu.SemaphoreType.DMA,
          pltpu.SemaphoreType.DMA,
      ],
      compiler_params=pltpu.CompilerParams(use_tc_tiling_on_sc=False),
  )


_sc_agg_first = _make_sc_kernel(True)
_sc_agg_next = _make_sc_kernel(False)


def _layer_norm_silu(h, g, be):
  mu = jnp.mean(h, axis=-1, keepdims=True)
  d = h - mu
  var = jnp.mean(d * d, axis=-1, keepdims=True)
  hn = d * lax.rsqrt(var + 1e-5) * g + be
  return hn * jax.nn.sigmoid(hn)


def _stage1_body(meanp_ref, x_ref, Wl_ref, Wr_ref, b_ref, g_ref, be_ref,
                 out_ref):
  mean = jnp.concatenate([meanp_ref[0], meanp_ref[1]], axis=-1)
  h = (jnp.dot(mean, Wl_ref[...], preferred_element_type=jnp.float32)
       + jnp.dot(x_ref[...], Wr_ref[...], preferred_element_type=jnp.float32)
       + b_ref[...])
  sil = _layer_norm_silu(h, g_ref[...], be_ref[...])
  out_ref[:, 0, :] = sil[:, :H]
  out_ref[:, 1, :] = sil[:, H:]


def _stage2_body(meanp_ref, hp_ref, Wl_ref, Wr_ref, b_ref, g_ref, be_ref,
                 Wm1_ref, bm1_ref, Wm2_ref, bm2_ref, out_ref):
  mean = jnp.concatenate([meanp_ref[0], meanp_ref[1]], axis=-1)
  hprev = jnp.concatenate([hp_ref[:, 0, :], hp_ref[:, 1, :]], axis=-1)
  h = (jnp.dot(mean, Wl_ref[...], preferred_element_type=jnp.float32)
       + jnp.dot(hprev, Wr_ref[...], preferred_element_type=jnp.float32)
       + b_ref[...])
  sil = _layer_norm_silu(h, g_ref[...], be_ref[...])
  m = jnp.maximum(
      jnp.dot(sil, Wm1_ref[...], preferred_element_type=jnp.float32)
      + bm1_ref[...], 0.0)
  out_ref[...] = (jnp.dot(m, Wm2_ref[...], preferred_element_type=jnp.float32)
                  + bm2_ref[...])


_half_spec = pl.BlockSpec((NC, BT, H), lambda i: (0, i, 0))
_full_spec = pl.BlockSpec((BT, D), lambda i: (i, 0))
_w_spec = pl.BlockSpec((D, D), lambda i: (0, 0))
_v_spec = pl.BlockSpec((1, D), lambda i: (0, 0))
_w2_spec = pl.BlockSpec((D, 1), lambda i: (0, 0))
_v2_spec = pl.BlockSpec((1, 1), lambda i: (0, 0))
_hp_spec = pl.BlockSpec((BT, 2, H), lambda i: (i, 0, 0))

_stage1 = pl.pallas_call(
    _stage1_body,
    grid=(G,),
    in_specs=[_half_spec, _full_spec, _w_spec, _w_spec,
              _v_spec, _v_spec, _v_spec],
    out_specs=_hp_spec,
    out_shape=jax.ShapeDtypeStruct((N, 2, H), jnp.float32),
)

_stage2 = pl.pallas_call(
    _stage2_body,
    grid=(G,),
    in_specs=[_half_spec, _hp_spec, _w_spec, _w_spec,
              _v_spec, _v_spec, _v_spec,
              _w_spec, _v_spec, _w2_spec, _v2_spec],
    out_specs=pl.BlockSpec((BT, 1), lambda i: (i, 0)),
    out_shape=jax.ShapeDtypeStruct((N, 1), jnp.float32),
)


@jax.jit
def kernel(x, edge_index, Wl0, Wr0, b0, g0, be0, Wl1, Wr1, b1, g1, be1,
           Wm1, bm1, Wm2, bm2):
  ei = edge_index.astype(jnp.int32)
  src2d = ei[0].reshape(NS, NCHUNK, C)
  dst2d = ei[1].reshape(NS, NCHUNK, C)

  meanp0, inv = _sc_agg_first(x.reshape(2 * N, H), src2d, dst2d)
  hp = _stage1(meanp0, x, Wl0, Wr0,
               b0.reshape(1, D), g0.reshape(1, D), be0.reshape(1, D))
  meanp1 = _sc_agg_next(hp.reshape(2 * N, H), src2d, dst2d, inv)
  return _stage2(meanp1, hp, Wl1, Wr1,
                 b1.reshape(1, D), g1.reshape(1, D), be1.reshape(1, D),
                 Wm1, bm1.reshape(1, D), Wm2, bm2.reshape(1, 1))
